# baseline (speedup 1.0000x reference)
import functools

import jax
import jax.numpy as jnp
from jax import lax
from jax.experimental import pallas as pl
from jax.experimental.pallas import tpu as pltpu
from jax.experimental.pallas import tpu_sc as plsc

N = 10000
E = 320000
D = 128
C = 10
G = 128

NC = 2
NS = 16
NW = NC * NS
E_PER_W = E // NW
CH = 128
NB = 80
E_PAD_W = NB * CH
NA = N + 8
RPS = 624
RPS_LAST = N - 15 * RPS


@functools.lru_cache(maxsize=None)
def _make_sc_agg():
    mesh = plsc.VectorSubcoreMesh(core_axis_name="c", subcore_axis_name="s")

    @functools.partial(
        pl.kernel,
        mesh=mesh,
        out_type=[jax.ShapeDtypeStruct((N, D), jnp.float32),
                  jax.ShapeDtypeStruct((N, D), jnp.float32)],
        scratch_types=[
            pltpu.VMEM((CH,), jnp.int32),
            pltpu.VMEM((CH,), jnp.int32),
            pltpu.VMEM((CH, D), jnp.float32),
            pltpu.VMEM_SHARED((NA, D), jnp.float32),
            pltpu.SemaphoreType.DMA,
        ],
    )
    def sc_agg(x_hbm, src_hbm, dst_hbm, zeros_hbm, out0, out1,
               idx_src, idx_dst, rows,
               acc, sem):
        cid = lax.axis_index("c")
        sid = lax.axis_index("s")

        r0 = sid * RPS

        @pl.when(sid < NS - 1)
        def _():
            pltpu.sync_copy(zeros_hbm.at[pl.ds(r0, RPS)],
                            acc.at[pl.ds(r0, RPS)])

        @pl.when(sid == NS - 1)
        def _():
            pltpu.sync_copy(zeros_hbm.at[pl.ds(15 * RPS, RPS_LAST)],
                            acc.at[pl.ds(15 * RPS, RPS_LAST)])

        plsc.subcore_barrier()

        w = cid * NS + sid
        wbase = w * E_PAD_W

        def body(t, carry):
            e0 = wbase + t * CH
            pltpu.sync_copy(src_hbm.at[pl.ds(e0, CH)], idx_src)
            pltpu.sync_copy(dst_hbm.at[pl.ds(e0, CH)], idx_dst)
            pltpu.async_copy(x_hbm.at[idx_src], rows, sem).wait()
            pltpu.sync_copy(rows, acc.at[idx_dst], add=True)
            return carry

        lax.fori_loop(0, NB, body, 0)

        plsc.subcore_barrier()

        @pl.when(jnp.logical_and(cid == 0, sid < NS - 1))
        def _():
            pltpu.sync_copy(acc.at[pl.ds(r0, RPS)], out0.at[pl.ds(r0, RPS)])

        @pl.when(jnp.logical_and(cid == 0, sid == NS - 1))
        def _():
            pltpu.sync_copy(acc.at[pl.ds(15 * RPS, RPS_LAST)],
                            out0.at[pl.ds(15 * RPS, RPS_LAST)])

        @pl.when(jnp.logical_and(cid == 1, sid < NS - 1))
        def _():
            pltpu.sync_copy(acc.at[pl.ds(r0, RPS)], out1.at[pl.ds(r0, RPS)])

        @pl.when(jnp.logical_and(cid == 1, sid == NS - 1))
        def _():
            pltpu.sync_copy(acc.at[pl.ds(15 * RPS, RPS_LAST)],
                            out1.at[pl.ds(15 * RPS, RPS_LAST)])

    return sc_agg


_R = 1000
_NBLK = N // _R


def _tc_layer_body(x_ref, p0_ref, p1_ref, ws_ref, wn_ref, b_ref, o_ref):
    x = x_ref[...]
    agg = p0_ref[...] + p1_ref[...]
    h = (jnp.dot(x, ws_ref[...], preferred_element_type=jnp.float32)
         + jnp.dot(agg, wn_ref[...], preferred_element_type=jnp.float32)
         + b_ref[...])
    o_ref[...] = jnp.maximum(h, 0.0)


def _tc_layer(x, p0, p1, w_self, w_neigh, b):
    return pl.pallas_call(
        _tc_layer_body,
        grid=(_NBLK,),
        in_specs=[
            pl.BlockSpec((_R, D), lambda i: (i, 0)),
            pl.BlockSpec((_R, D), lambda i: (i, 0)),
            pl.BlockSpec((_R, D), lambda i: (i, 0)),
            pl.BlockSpec((D, D), lambda i: (0, 0)),
            pl.BlockSpec((D, D), lambda i: (0, 0)),
            pl.BlockSpec((1, D), lambda i: (0, 0)),
        ],
        out_specs=pl.BlockSpec((_R, D), lambda i: (i, 0)),
        out_shape=jax.ShapeDtypeStruct((N, D), jnp.float32),
    )(x, p0, p1, w_self, w_neigh, b.reshape(1, D))


def _tc_final_body(x_ref, p0_ref, p1_ref, ws_ref, wn_ref, b_ref,
                   batch_ref, wd1_ref, bd1_ref, wd2_ref, bd2_ref,
                   o_ref, acc_sum, acc_cnt):
    i = pl.program_id(0)

    @pl.when(i == 0)
    def _():
        acc_sum[...] = jnp.zeros_like(acc_sum)
        acc_cnt[...] = jnp.zeros_like(acc_cnt)

    x = x_ref[...]
    agg = p0_ref[...] + p1_ref[...]
    h = (jnp.dot(x, ws_ref[...], preferred_element_type=jnp.float32)
         + jnp.dot(agg, wn_ref[...], preferred_element_type=jnp.float32)
         + b_ref[...])
    h = jnp.maximum(h, 0.0)

    labels = batch_ref[...].reshape(1, _R)
    onehot_t = (labels == lax.broadcasted_iota(jnp.int32, (G, _R), 0))
    onehot_t = onehot_t.astype(jnp.float32)
    acc_sum[...] += jnp.dot(onehot_t, h, preferred_element_type=jnp.float32)
    acc_cnt[...] += jnp.dot(onehot_t, jnp.ones((_R, D), jnp.float32),
                            preferred_element_type=jnp.float32)

    @pl.when(i == _NBLK - 1)
    def _():
        pooled = acc_sum[...] / jnp.maximum(acc_cnt[...], 1.0)
        t = jnp.maximum(
            jnp.dot(pooled, wd1_ref[...], preferred_element_type=jnp.float32)
            + bd1_ref[...], 0.0)
        logits = (jnp.dot(t, wd2_ref[...], preferred_element_type=jnp.float32)
                  + bd2_ref[...])
        m = jnp.max(logits, axis=-1, keepdims=True)
        e = jnp.exp(logits - m)
        o_ref[...] = e / jnp.sum(e, axis=-1, keepdims=True)


def _tc_final(h1, p0, p1, w_self, w_neigh, b, batch, wd1, bd1, wd2, bd2):
    return pl.pallas_call(
        _tc_final_body,
        grid=(_NBLK,),
        in_specs=[
            pl.BlockSpec((_R, D), lambda i: (i, 0)),
            pl.BlockSpec((_R, D), lambda i: (i, 0)),
            pl.BlockSpec((_R, D), lambda i: (i, 0)),
            pl.BlockSpec((D, D), lambda i: (0, 0)),
            pl.BlockSpec((D, D), lambda i: (0, 0)),
            pl.BlockSpec((1, D), lambda i: (0, 0)),
            pl.BlockSpec((1, 1, _R), lambda i: (i, 0, 0)),
            pl.BlockSpec((D, D), lambda i: (0, 0)),
            pl.BlockSpec((1, D), lambda i: (0, 0)),
            pl.BlockSpec((D, C), lambda i: (0, 0)),
            pl.BlockSpec((1, C), lambda i: (0, 0)),
        ],
        out_specs=pl.BlockSpec((G, C), lambda i: (0, 0)),
        out_shape=jax.ShapeDtypeStruct((G, C), jnp.float32),
        scratch_shapes=[
            pltpu.VMEM((G, D), jnp.float32),
            pltpu.VMEM((G, D), jnp.float32),
        ],
    )(h1, p0, p1, w_self, w_neigh, b.reshape(1, D),
      batch.reshape(_NBLK, 1, _R), wd1, bd1.reshape(1, D), wd2,
      bd2.reshape(1, C))


def kernel(node_embeddings, edge_index, batch, W1_self, W1_neigh, b1,
           W2_self, W2_neigh, b2, Wd1, bd1, Wd2, bd2):
    n_pad = E_PAD_W - E_PER_W
    src = jnp.concatenate(
        [edge_index[0].reshape(NW, E_PER_W),
         jnp.zeros((NW, n_pad), jnp.int32)], axis=1).reshape(NW * E_PAD_W)
    trash = N + (jnp.arange(n_pad, dtype=jnp.int32) % 8)
    dst = jnp.concatenate(
        [edge_index[1].reshape(NW, E_PER_W),
         jnp.broadcast_to(trash, (NW, n_pad))], axis=1).reshape(NW * E_PAD_W)
    zeros = jnp.zeros((N, D), jnp.float32)

    sc_agg = _make_sc_agg()
    p0, p1 = sc_agg(node_embeddings, src, dst, zeros)
    h1 = _tc_layer(node_embeddings, p0, p1, W1_self, W1_neigh, b1)
    q0, q1 = sc_agg(h1, src, dst, zeros)
    return _tc_final(h1, q0, q1, W2_self, W2_neigh, b2, batch,
                     Wd1, bd1, Wd2, bd2)


# speedup vs baseline: 1.1342x; 1.1342x over previous
import functools

import jax
import jax.numpy as jnp
from jax import lax
from jax.experimental import pallas as pl
from jax.experimental.pallas import tpu as pltpu
from jax.experimental.pallas import tpu_sc as plsc

N = 10000
E = 320000
D = 128
C = 10
G = 128

NC = 2
NS = 16
NW = NC * NS
E_PER_W = E // NW
CH = 128
NB = 80
E_PAD_W = NB * CH
NA = N + 8 * NW
RPS = 624
RPS_LAST = N - 15 * RPS


@functools.lru_cache(maxsize=None)
def _make_sc_agg():
    mesh = plsc.VectorSubcoreMesh(core_axis_name="c", subcore_axis_name="s")

    @functools.partial(
        pl.kernel,
        mesh=mesh,
        out_type=[jax.ShapeDtypeStruct((N, D), jnp.float32),
                  jax.ShapeDtypeStruct((N, D), jnp.float32)],
        scratch_types=[
            pltpu.VMEM((NB, CH), jnp.int32),
            pltpu.VMEM((NB, CH), jnp.int32),
            pltpu.VMEM((CH, D), jnp.float32),
            pltpu.VMEM_SHARED((NA, D), jnp.float32),
            pltpu.SemaphoreType.DMA,
        ],
    )
    def sc_agg(x_hbm, src_hbm, dst_hbm, zeros_hbm, out0, out1,
               tab_src, tab_dst, rows,
               acc, sem):
        cid = lax.axis_index("c")
        sid = lax.axis_index("s")

        r0 = sid * RPS

        @pl.when(sid < NS - 1)
        def _():
            pltpu.sync_copy(zeros_hbm.at[pl.ds(r0, RPS)],
                            acc.at[pl.ds(r0, RPS)])

        @pl.when(sid == NS - 1)
        def _():
            pltpu.sync_copy(zeros_hbm.at[pl.ds(15 * RPS, RPS_LAST)],
                            acc.at[pl.ds(15 * RPS, RPS_LAST)])

        plsc.subcore_barrier()

        w = cid * NS + sid
        pltpu.sync_copy(src_hbm.at[pl.ds(w * NB, NB)], tab_src)
        pltpu.sync_copy(dst_hbm.at[pl.ds(w * NB, NB)], tab_dst)

        def body(t, carry):
            pltpu.async_copy(x_hbm.at[tab_src.at[t]], rows, sem).wait()
            pltpu.sync_copy(rows, acc.at[tab_dst.at[t]], add=True)
            return carry

        lax.fori_loop(0, NB, body, 0)

        plsc.subcore_barrier()

        @pl.when(jnp.logical_and(cid == 0, sid < NS - 1))
        def _():
            pltpu.sync_copy(acc.at[pl.ds(r0, RPS)], out0.at[pl.ds(r0, RPS)])

        @pl.when(jnp.logical_and(cid == 0, sid == NS - 1))
        def _():
            pltpu.sync_copy(acc.at[pl.ds(15 * RPS, RPS_LAST)],
                            out0.at[pl.ds(15 * RPS, RPS_LAST)])

        @pl.when(jnp.logical_and(cid == 1, sid < NS - 1))
        def _():
            pltpu.sync_copy(acc.at[pl.ds(r0, RPS)], out1.at[pl.ds(r0, RPS)])

        @pl.when(jnp.logical_and(cid == 1, sid == NS - 1))
        def _():
            pltpu.sync_copy(acc.at[pl.ds(15 * RPS, RPS_LAST)],
                            out1.at[pl.ds(15 * RPS, RPS_LAST)])

    return sc_agg


_R = 1000
_NBLK = N // _R


def _tc_layer_body(x_ref, p0_ref, p1_ref, ws_ref, wn_ref, b_ref, o_ref):
    x = x_ref[...]
    agg = p0_ref[...] + p1_ref[...]
    h = (jnp.dot(x, ws_ref[...], preferred_element_type=jnp.float32)
         + jnp.dot(agg, wn_ref[...], preferred_element_type=jnp.float32)
         + b_ref[...])
    o_ref[...] = jnp.maximum(h, 0.0)


def _tc_layer(x, p0, p1, w_self, w_neigh, b):
    return pl.pallas_call(
        _tc_layer_body,
        grid=(_NBLK,),
        in_specs=[
            pl.BlockSpec((_R, D), lambda i: (i, 0)),
            pl.BlockSpec((_R, D), lambda i: (i, 0)),
            pl.BlockSpec((_R, D), lambda i: (i, 0)),
            pl.BlockSpec((D, D), lambda i: (0, 0)),
            pl.BlockSpec((D, D), lambda i: (0, 0)),
            pl.BlockSpec((1, D), lambda i: (0, 0)),
        ],
        out_specs=pl.BlockSpec((_R, D), lambda i: (i, 0)),
        out_shape=jax.ShapeDtypeStruct((N, D), jnp.float32),
    )(x, p0, p1, w_self, w_neigh, b.reshape(1, D))


def _tc_final_body(x_ref, p0_ref, p1_ref, ws_ref, wn_ref, b_ref,
                   batch_ref, wd1_ref, bd1_ref, wd2_ref, bd2_ref,
                   o_ref, acc_sum, acc_cnt):
    i = pl.program_id(0)

    @pl.when(i == 0)
    def _():
        acc_sum[...] = jnp.zeros_like(acc_sum)
        acc_cnt[...] = jnp.zeros_like(acc_cnt)

    x = x_ref[...]
    agg = p0_ref[...] + p1_ref[...]
    h = (jnp.dot(x, ws_ref[...], preferred_element_type=jnp.float32)
         + jnp.dot(agg, wn_ref[...], preferred_element_type=jnp.float32)
         + b_ref[...])
    h = jnp.maximum(h, 0.0)

    labels = batch_ref[...].reshape(1, _R)
    onehot_t = (labels == lax.broadcasted_iota(jnp.int32, (G, _R), 0))
    onehot_t = onehot_t.astype(jnp.float32)
    acc_sum[...] += jnp.dot(onehot_t, h, preferred_element_type=jnp.float32)
    acc_cnt[...] += jnp.dot(onehot_t, jnp.ones((_R, D), jnp.float32),
                            preferred_element_type=jnp.float32)

    @pl.when(i == _NBLK - 1)
    def _():
        pooled = acc_sum[...] / jnp.maximum(acc_cnt[...], 1.0)
        t = jnp.maximum(
            jnp.dot(pooled, wd1_ref[...], preferred_element_type=jnp.float32)
            + bd1_ref[...], 0.0)
        logits = (jnp.dot(t, wd2_ref[...], preferred_element_type=jnp.float32)
                  + bd2_ref[...])
        m = jnp.max(logits, axis=-1, keepdims=True)
        e = jnp.exp(logits - m)
        o_ref[...] = e / jnp.sum(e, axis=-1, keepdims=True)


def _tc_final(h1, p0, p1, w_self, w_neigh, b, batch, wd1, bd1, wd2, bd2):
    return pl.pallas_call(
        _tc_final_body,
        grid=(_NBLK,),
        in_specs=[
            pl.BlockSpec((_R, D), lambda i: (i, 0)),
            pl.BlockSpec((_R, D), lambda i: (i, 0)),
            pl.BlockSpec((_R, D), lambda i: (i, 0)),
            pl.BlockSpec((D, D), lambda i: (0, 0)),
            pl.BlockSpec((D, D), lambda i: (0, 0)),
            pl.BlockSpec((1, D), lambda i: (0, 0)),
            pl.BlockSpec((1, 1, _R), lambda i: (i, 0, 0)),
            pl.BlockSpec((D, D), lambda i: (0, 0)),
            pl.BlockSpec((1, D), lambda i: (0, 0)),
            pl.BlockSpec((D, C), lambda i: (0, 0)),
            pl.BlockSpec((1, C), lambda i: (0, 0)),
        ],
        out_specs=pl.BlockSpec((G, C), lambda i: (0, 0)),
        out_shape=jax.ShapeDtypeStruct((G, C), jnp.float32),
        scratch_shapes=[
            pltpu.VMEM((G, D), jnp.float32),
            pltpu.VMEM((G, D), jnp.float32),
        ],
    )(h1, p0, p1, w_self, w_neigh, b.reshape(1, D),
      batch.reshape(_NBLK, 1, _R), wd1, bd1.reshape(1, D), wd2,
      bd2.reshape(1, C))


def kernel(node_embeddings, edge_index, batch, W1_self, W1_neigh, b1,
           W2_self, W2_neigh, b2, Wd1, bd1, Wd2, bd2):
    n_pad = E_PAD_W - E_PER_W
    src = jnp.concatenate(
        [edge_index[0].reshape(NW, E_PER_W),
         jnp.zeros((NW, n_pad), jnp.int32)], axis=1).reshape(NW * NB, CH)
    trash = (N + 8 * jnp.arange(NW, dtype=jnp.int32)[:, None]
             + (jnp.arange(n_pad, dtype=jnp.int32) % 8)[None, :])
    dst = jnp.concatenate(
        [edge_index[1].reshape(NW, E_PER_W), trash],
        axis=1).reshape(NW * NB, CH)
    zeros = jnp.zeros((N, D), jnp.float32)

    sc_agg = _make_sc_agg()
    p0, p1 = sc_agg(node_embeddings, src, dst, zeros)
    h1 = _tc_layer(node_embeddings, p0, p1, W1_self, W1_neigh, b1)
    q0, q1 = sc_agg(h1, src, dst, zeros)
    return _tc_final(h1, q0, q1, W2_self, W2_neigh, b2, batch,
                     Wd1, bd1, Wd2, bd2)


# speedup vs baseline: 2.6741x; 2.3578x over previous
import functools

import jax
import jax.numpy as jnp
from jax import lax
from jax.experimental import pallas as pl
from jax.experimental.pallas import tpu as pltpu
from jax.experimental.pallas import tpu_sc as plsc

N = 10000
E = 320000
D = 128
C = 10
G = 128

NC = 2
NS = 16
NW = NC * NS
E_PER_W = E // NW
CH = 128
NB = 80
E_PAD_W = NB * CH
NA = N + 8 * NW
RPS = 624
RPS_LAST = N - 15 * RPS


@functools.lru_cache(maxsize=None)
def _make_sc_agg():
    mesh = plsc.VectorSubcoreMesh(core_axis_name="c", subcore_axis_name="s")

    @functools.partial(
        pl.kernel,
        mesh=mesh,
        out_type=[jax.ShapeDtypeStruct((N, D), jnp.float32),
                  jax.ShapeDtypeStruct((N, D), jnp.float32)],
        scratch_types=[
            pltpu.VMEM((NB, CH), jnp.int32),
            pltpu.VMEM((NB, CH), jnp.int32),
            pltpu.VMEM((CH, D), jnp.float32),
            pltpu.VMEM_SHARED((NA, D), jnp.float32),
            pltpu.SemaphoreType.DMA,
        ],
    )
    def sc_agg(x_hbm, src_hbm, dst_hbm, zeros_hbm, out0, out1,
               tab_src, tab_dst, rows,
               acc, sem):
        cid = lax.axis_index("c")
        sid = lax.axis_index("s")

        r0 = sid * RPS

        @pl.when(sid < NS - 1)
        def _():
            pltpu.sync_copy(zeros_hbm.at[pl.ds(r0, RPS)],
                            acc.at[pl.ds(r0, RPS)])

        @pl.when(sid == NS - 1)
        def _():
            pltpu.sync_copy(zeros_hbm.at[pl.ds(15 * RPS, RPS_LAST)],
                            acc.at[pl.ds(15 * RPS, RPS_LAST)])

        plsc.subcore_barrier()

        w = cid * NS + sid
        pltpu.sync_copy(src_hbm.at[pl.ds(w * NB, NB)], tab_src)
        pltpu.sync_copy(dst_hbm.at[pl.ds(w * NB, NB)], tab_dst)

        def body(t, carry):
            pltpu.async_copy(x_hbm.at[tab_src.at[t]], rows, sem).wait()
            pltpu.sync_copy(rows, acc.at[tab_dst.at[t]], add=True)
            return carry

        lax.fori_loop(0, NB, body, 0)

        plsc.subcore_barrier()

        @pl.when(jnp.logical_and(cid == 0, sid < NS - 1))
        def _():
            pltpu.sync_copy(acc.at[pl.ds(r0, RPS)], out0.at[pl.ds(r0, RPS)])

        @pl.when(jnp.logical_and(cid == 0, sid == NS - 1))
        def _():
            pltpu.sync_copy(acc.at[pl.ds(15 * RPS, RPS_LAST)],
                            out0.at[pl.ds(15 * RPS, RPS_LAST)])

        @pl.when(jnp.logical_and(cid == 1, sid < NS - 1))
        def _():
            pltpu.sync_copy(acc.at[pl.ds(r0, RPS)], out1.at[pl.ds(r0, RPS)])

        @pl.when(jnp.logical_and(cid == 1, sid == NS - 1))
        def _():
            pltpu.sync_copy(acc.at[pl.ds(15 * RPS, RPS_LAST)],
                            out1.at[pl.ds(15 * RPS, RPS_LAST)])

    return sc_agg


_R = 1000
_NBLK = N // _R


def _tc_layer_body(x_ref, p0_ref, p1_ref, ws_ref, wn_ref, b_ref, o_ref):
    x = x_ref[...]
    agg = p0_ref[...] + p1_ref[...]
    h = (jnp.dot(x, ws_ref[...], preferred_element_type=jnp.float32)
         + jnp.dot(agg, wn_ref[...], preferred_element_type=jnp.float32)
         + b_ref[...])
    o_ref[...] = jnp.maximum(h, 0.0)


def _tc_layer(x, p0, p1, w_self, w_neigh, b):
    return pl.pallas_call(
        _tc_layer_body,
        grid=(_NBLK,),
        in_specs=[
            pl.BlockSpec((_R, D), lambda i: (i, 0)),
            pl.BlockSpec((_R, D), lambda i: (i, 0)),
            pl.BlockSpec((_R, D), lambda i: (i, 0)),
            pl.BlockSpec((D, D), lambda i: (0, 0)),
            pl.BlockSpec((D, D), lambda i: (0, 0)),
            pl.BlockSpec((1, D), lambda i: (0, 0)),
        ],
        out_specs=pl.BlockSpec((_R, D), lambda i: (i, 0)),
        out_shape=jax.ShapeDtypeStruct((N, D), jnp.float32),
    )(x, p0, p1, w_self, w_neigh, b.reshape(1, D))


def _tc_final_body(x_ref, p0_ref, p1_ref, ws_ref, wn_ref, b_ref,
                   batch_ref, wd1_ref, bd1_ref, wd2_ref, bd2_ref,
                   o_ref, acc_sum, acc_cnt):
    i = pl.program_id(0)

    @pl.when(i == 0)
    def _():
        acc_sum[...] = jnp.zeros_like(acc_sum)
        acc_cnt[...] = jnp.zeros_like(acc_cnt)

    x = x_ref[...]
    agg = p0_ref[...] + p1_ref[...]
    h = (jnp.dot(x, ws_ref[...], preferred_element_type=jnp.float32)
         + jnp.dot(agg, wn_ref[...], preferred_element_type=jnp.float32)
         + b_ref[...])
    h = jnp.maximum(h, 0.0)

    labels = batch_ref[...].reshape(1, _R)
    onehot_t = (labels == lax.broadcasted_iota(jnp.int32, (G, _R), 0))
    onehot_t = onehot_t.astype(jnp.float32)
    acc_sum[...] += jnp.dot(onehot_t, h, preferred_element_type=jnp.float32)
    acc_cnt[...] += jnp.dot(onehot_t, jnp.ones((_R, D), jnp.float32),
                            preferred_element_type=jnp.float32)

    @pl.when(i == _NBLK - 1)
    def _():
        pooled = acc_sum[...] / jnp.maximum(acc_cnt[...], 1.0)
        t = jnp.maximum(
            jnp.dot(pooled, wd1_ref[...], preferred_element_type=jnp.float32)
            + bd1_ref[...], 0.0)
        logits = (jnp.dot(t, wd2_ref[...], preferred_element_type=jnp.float32)
                  + bd2_ref[...])
        m = jnp.max(logits, axis=-1, keepdims=True)
        e = jnp.exp(logits - m)
        o_ref[...] = e / jnp.sum(e, axis=-1, keepdims=True)


def _tc_final(h1, p0, p1, w_self, w_neigh, b, batch, wd1, bd1, wd2, bd2):
    return pl.pallas_call(
        _tc_final_body,
        grid=(_NBLK,),
        in_specs=[
            pl.BlockSpec((_R, D), lambda i: (i, 0)),
            pl.BlockSpec((_R, D), lambda i: (i, 0)),
            pl.BlockSpec((_R, D), lambda i: (i, 0)),
            pl.BlockSpec((D, D), lambda i: (0, 0)),
            pl.BlockSpec((D, D), lambda i: (0, 0)),
            pl.BlockSpec((1, D), lambda i: (0, 0)),
            pl.BlockSpec((1, 1, _R), lambda i: (i, 0, 0)),
            pl.BlockSpec((D, D), lambda i: (0, 0)),
            pl.BlockSpec((1, D), lambda i: (0, 0)),
            pl.BlockSpec((D, C), lambda i: (0, 0)),
            pl.BlockSpec((1, C), lambda i: (0, 0)),
        ],
        out_specs=pl.BlockSpec((G, C), lambda i: (0, 0)),
        out_shape=jax.ShapeDtypeStruct((G, C), jnp.float32),
        scratch_shapes=[
            pltpu.VMEM((G, D), jnp.float32),
            pltpu.VMEM((G, D), jnp.float32),
        ],
    )(h1, p0, p1, w_self, w_neigh, b.reshape(1, D),
      batch.reshape(_NBLK, 1, _R), wd1, bd1.reshape(1, D), wd2,
      bd2.reshape(1, C))


def kernel(node_embeddings, edge_index, batch, W1_self, W1_neigh, b1,
           W2_self, W2_neigh, b2, Wd1, bd1, Wd2, bd2):
    n_pad = E_PAD_W - E_PER_W
    src2 = edge_index[0].reshape(NW, E_PER_W)
    src = jnp.concatenate(
        [src2, src2[:, :n_pad]], axis=1).reshape(NW * NB, CH)
    trash = (N + 8 * jnp.arange(NW, dtype=jnp.int32)[:, None]
             + (jnp.arange(n_pad, dtype=jnp.int32) % 8)[None, :])
    dst = jnp.concatenate(
        [edge_index[1].reshape(NW, E_PER_W), trash],
        axis=1).reshape(NW * NB, CH)
    zeros = jnp.zeros((N, D), jnp.float32)

    sc_agg = _make_sc_agg()
    p0, p1 = sc_agg(node_embeddings, src, dst, zeros)
    h1 = _tc_layer(node_embeddings, p0, p1, W1_self, W1_neigh, b1)
    q0, q1 = sc_agg(h1, src, dst, zeros)
    return _tc_final(h1, q0, q1, W2_self, W2_neigh, b2, batch,
                     Wd1, bd1, Wd2, bd2)
